# Initial kernel scaffold; baseline (speedup 1.0000x reference)
import functools

import jax
import jax.numpy as jnp
from jax import lax
from jax.experimental import pallas as pl
from jax.experimental.pallas import tpu as pltpu, tpu_sc as plsc

N = 10000
E = 320000
D = 128
DG = 16
NC = 2
NS = 16
NW = NC * NS
CH = 128
NCH = 79
EPW = NCH * CH
NPAD = 10240
RPT = NPAD // NS
BLK = 512
GRID = NPAD // BLK

_mesh = plsc.VectorSubcoreMesh(core_axis_name="c", subcore_axis_name="s")


def _sc_body(h_hbm, srcs_hbm, dsts_hbm, zd_hbm, zg_hbm, og_hbm,
             msg_out, deg_out, src_v, dst_v, rows_v, ones_v, acc_sh, dacc_sh,
             *, with_deg):
    c = lax.axis_index("c")
    s = lax.axis_index("s")
    pltpu.sync_copy(srcs_hbm.at[c, s], src_v)
    pltpu.sync_copy(dsts_hbm.at[c, s], dst_v)
    pltpu.sync_copy(zd_hbm, acc_sh.at[pl.ds(s * RPT, RPT)])
    if with_deg:
        pltpu.sync_copy(og_hbm, ones_v)
        pltpu.sync_copy(zg_hbm, dacc_sh.at[pl.ds(s * RPT, RPT)])
    plsc.subcore_barrier()

    def body(j, carry):
        pltpu.sync_copy(h_hbm.at[src_v.at[j]], rows_v)
        pltpu.sync_copy(rows_v, acc_sh.at[dst_v.at[j]], add=True)
        if with_deg:
            pltpu.sync_copy(ones_v, dacc_sh.at[dst_v.at[j]], add=True)
        return carry

    lax.fori_loop(0, NCH, body, 0)
    plsc.subcore_barrier()
    pltpu.sync_copy(acc_sh.at[pl.ds(s * RPT, RPT)],
                    msg_out.at[c, pl.ds(s * RPT, RPT)])
    if with_deg:
        pltpu.sync_copy(dacc_sh.at[pl.ds(s * RPT, RPT)],
                        deg_out.at[c, pl.ds(s * RPT, RPT)])


def _make_sc(with_deg):
    out_type = (jax.ShapeDtypeStruct((NC, NPAD, D), jnp.float32),)
    if with_deg:
        out_type = out_type + (jax.ShapeDtypeStruct((NC, NPAD, DG), jnp.float32),)
    scratch = [
        pltpu.VMEM((NCH, CH), jnp.int32),
        pltpu.VMEM((NCH, CH), jnp.int32),
        pltpu.VMEM((CH, D), jnp.float32),
        pltpu.VMEM((CH, DG), jnp.float32),
        pltpu.VMEM_SHARED((NPAD, D), jnp.float32),
        pltpu.VMEM_SHARED((NPAD, DG), jnp.float32),
    ]

    if with_deg:
        def body(h, srcs, dsts, zd, zg, og, msg_out, deg_out,
                 src_v, dst_v, rows_v, ones_v, acc_sh, dacc_sh):
            _sc_body(h, srcs, dsts, zd, zg, og, msg_out, deg_out,
                     src_v, dst_v, rows_v, ones_v, acc_sh, dacc_sh,
                     with_deg=True)
    else:
        def body(h, srcs, dsts, zd, zg, og, msg_out,
                 src_v, dst_v, rows_v, ones_v, acc_sh, dacc_sh):
            _sc_body(h, srcs, dsts, zd, zg, og, msg_out, None,
                     src_v, dst_v, rows_v, ones_v, acc_sh, dacc_sh,
                     with_deg=False)

    return pl.kernel(body, out_type=out_type, mesh=_mesh, scratch_types=scratch)


_sc_seg_deg = _make_sc(True)
_sc_seg = _make_sc(False)


def _tc0_body(h, m0, m1, g0, g1, ws, wn, bias, out, invb):
    deg = g0[:, 0:1] + g1[:, 0:1]
    inv = 1.0 / jnp.maximum(deg, 1.0)
    invb[...] = jnp.broadcast_to(inv, (BLK, D))
    msg = (m0[...] + m1[...]) * inv
    r = (jnp.dot(h[...], ws[...], preferred_element_type=jnp.float32)
         + jnp.dot(msg, wn[...], preferred_element_type=jnp.float32)
         + bias[...])
    out[...] = jnp.maximum(r, 0.0)


def _make_tc12(relu):
    def body(h, m0, m1, invb, ws, wn, bias, out):
        msg = (m0[...] + m1[...]) * invb[...]
        r = (jnp.dot(h[...], ws[...], preferred_element_type=jnp.float32)
             + jnp.dot(msg, wn[...], preferred_element_type=jnp.float32)
             + bias[...])
        out[...] = jnp.maximum(r, 0.0) if relu else r
    return body


_row_spec = pl.BlockSpec((BLK, D), lambda i: (i, 0))
_deg_spec = pl.BlockSpec((BLK, DG), lambda i: (i, 0))
_w_spec = pl.BlockSpec((D, D), lambda i: (0, 0))
_b_spec = pl.BlockSpec((1, D), lambda i: (0, 0))


def _tc_layer0(h, m0, m1, g0, g1, ws, wn, bias):
    return pl.pallas_call(
        _tc0_body,
        grid=(GRID,),
        in_specs=[_row_spec, _row_spec, _row_spec, _deg_spec, _deg_spec,
                  _w_spec, _w_spec, _b_spec],
        out_specs=[_row_spec, _row_spec],
        out_shape=[jax.ShapeDtypeStruct((NPAD, D), jnp.float32),
                   jax.ShapeDtypeStruct((NPAD, D), jnp.float32)],
    )(h, m0, m1, g0, g1, ws, wn, bias)


def _tc_layer12(h, m0, m1, invb, ws, wn, bias, relu):
    return pl.pallas_call(
        _make_tc12(relu),
        grid=(GRID,),
        in_specs=[_row_spec, _row_spec, _row_spec, _row_spec,
                  _w_spec, _w_spec, _b_spec],
        out_specs=_row_spec,
        out_shape=jax.ShapeDtypeStruct((NPAD, D), jnp.float32),
    )(h, m0, m1, invb, ws, wn, bias)


def kernel(x, edge_index, W_self_0, W_neigh_0, b_0, W_self_1, W_neigh_1, b_1,
           W_self_2, W_neigh_2, b_2):
    src = edge_index[0].astype(jnp.int32)
    dst = edge_index[1].astype(jnp.int32)
    pad_n = NW * EPW - E
    pad_src = jnp.zeros((pad_n,), jnp.int32)
    pad_dst = (N + jnp.arange(pad_n, dtype=jnp.int32) % (NPAD - N)).astype(jnp.int32)
    srcs = jnp.concatenate([src, pad_src]).reshape(NC, NS, NCH, CH)
    dsts = jnp.concatenate([dst, pad_dst]).reshape(NC, NS, NCH, CH)

    zd = jnp.zeros((RPT, D), jnp.float32)
    zg = jnp.zeros((RPT, DG), jnp.float32)
    og = jnp.ones((CH, DG), jnp.float32)

    h = jnp.pad(x, ((0, NPAD - N), (0, 0)))
    bias0 = b_0.reshape(1, D)
    bias1 = b_1.reshape(1, D)
    bias2 = b_2.reshape(1, D)

    msg, degp = _sc_seg_deg(h, srcs, dsts, zd, zg, og)
    h, invb = _tc_layer0(h, msg[0], msg[1], degp[0], degp[1],
                         W_self_0, W_neigh_0, bias0)

    (msg,) = _sc_seg(h, srcs, dsts, zd, zg, og)
    h = _tc_layer12(h, msg[0], msg[1], invb, W_self_1, W_neigh_1, bias1,
                    relu=True)

    (msg,) = _sc_seg(h, srcs, dsts, zd, zg, og)
    h = _tc_layer12(h, msg[0], msg[1], invb, W_self_2, W_neigh_2, bias2,
                    relu=False)
    return h[:N]


# speedup vs baseline: 3.1969x; 3.1969x over previous
import functools

import jax
import jax.numpy as jnp
from jax import lax
from jax.experimental import pallas as pl
from jax.experimental.pallas import tpu as pltpu, tpu_sc as plsc

N = 10000
E = 320000
D = 128
DG = 16
NC = 2
NS = 16
NW = NC * NS
CH = 128
NCH = 79
EPW = NCH * CH
NPAD = 10240
RPT = NPAD // NS
BLK = 512
GRID = NPAD // BLK


@functools.cache
def _get_mesh():
    return plsc.VectorSubcoreMesh(core_axis_name="c", subcore_axis_name="s",
                                  num_cores=NC, num_subcores=NS)


def _seg_body(h_hbm, srcs_hbm, dsts_hbm, zd_hbm, msg_out,
              src_v, dst_v, rows_v, acc_sh):
    c = lax.axis_index("c")
    s = lax.axis_index("s")
    pltpu.sync_copy(srcs_hbm.at[c, s], src_v)
    pltpu.sync_copy(dsts_hbm.at[c, s], dst_v)
    pltpu.sync_copy(zd_hbm, acc_sh.at[pl.ds(s * RPT, RPT)])
    plsc.subcore_barrier()

    def body(j, carry):
        pltpu.sync_copy(h_hbm.at[src_v.at[j]], rows_v)
        pltpu.sync_copy(rows_v, acc_sh.at[dst_v.at[j]], add=True)
        return carry

    lax.fori_loop(0, NCH, body, 0)
    plsc.subcore_barrier()
    pltpu.sync_copy(acc_sh.at[pl.ds(s * RPT, RPT)],
                    msg_out.at[c, pl.ds(s * RPT, RPT)])


@functools.cache
def _make_seg():
    return pl.kernel(
        _seg_body,
        out_type=jax.ShapeDtypeStruct((NC, NPAD, D), jnp.float32),
        mesh=_get_mesh(),
        scratch_types=[
            pltpu.VMEM((NCH, CH), jnp.int32),
            pltpu.VMEM((NCH, CH), jnp.int32),
            pltpu.VMEM((CH, D), jnp.float32),
            pltpu.VMEM_SHARED((NPAD, D), jnp.float32),
        ])


def _tc0_body(h, m0, m1, g0, g1, ws, wn, bias, out, invb):
    deg = g0[:, 0:1] + g1[:, 0:1]
    inv = 1.0 / jnp.maximum(deg, 1.0)
    invb[...] = jnp.broadcast_to(inv, (BLK, DG))
    msg = (m0[...] + m1[...]) * inv
    r = (jnp.dot(h[...], ws[...], preferred_element_type=jnp.float32)
         + jnp.dot(msg, wn[...], preferred_element_type=jnp.float32)
         + bias[...])
    out[...] = jnp.maximum(r, 0.0)


def _make_tc12(relu):
    def body(h, m0, m1, invb, ws, wn, bias, out):
        msg = (m0[...] + m1[...]) * invb[:, 0:1]
        r = (jnp.dot(h[...], ws[...], preferred_element_type=jnp.float32)
             + jnp.dot(msg, wn[...], preferred_element_type=jnp.float32)
             + bias[...])
        out[...] = jnp.maximum(r, 0.0) if relu else r
    return body


_row_spec = pl.BlockSpec((BLK, D), lambda i: (i, 0))
_deg_spec = pl.BlockSpec((BLK, DG), lambda i: (i, 0))
_w_spec = pl.BlockSpec((D, D), lambda i: (0, 0))
_b_spec = pl.BlockSpec((1, D), lambda i: (0, 0))


def _tc_layer0(h, m0, m1, g0, g1, ws, wn, bias):
    return pl.pallas_call(
        _tc0_body,
        grid=(GRID,),
        in_specs=[_row_spec, _row_spec, _row_spec, _row_spec, _row_spec,
                  _w_spec, _w_spec, _b_spec],
        out_specs=[_row_spec, _deg_spec],
        out_shape=[jax.ShapeDtypeStruct((NPAD, D), jnp.float32),
                   jax.ShapeDtypeStruct((NPAD, DG), jnp.float32)],
    )(h, m0, m1, g0, g1, ws, wn, bias)


def _tc_layer12(h, m0, m1, invb, ws, wn, bias, relu):
    return pl.pallas_call(
        _make_tc12(relu),
        grid=(GRID,),
        in_specs=[_row_spec, _row_spec, _row_spec, _deg_spec,
                  _w_spec, _w_spec, _b_spec],
        out_specs=_row_spec,
        out_shape=jax.ShapeDtypeStruct((NPAD, D), jnp.float32),
    )(h, m0, m1, invb, ws, wn, bias)


def kernel(x, edge_index, W_self_0, W_neigh_0, b_0, W_self_1, W_neigh_1, b_1,
           W_self_2, W_neigh_2, b_2):
    src = edge_index[0].astype(jnp.int32)
    dst = edge_index[1].astype(jnp.int32)
    pad_n = NW * EPW - E
    pad_src = jnp.zeros((pad_n,), jnp.int32)
    pad_dst = (N + jnp.arange(pad_n, dtype=jnp.int32) % (NPAD - N)).astype(jnp.int32)
    srcs = jnp.concatenate([src, pad_src]).reshape(NC, NS, NCH, CH)
    dsts = jnp.concatenate([dst, pad_dst]).reshape(NC, NS, NCH, CH)

    zd = jnp.zeros((RPT, D), jnp.float32)
    ones_tab = jnp.ones((NPAD, D), jnp.float32)

    h = jnp.pad(x, ((0, NPAD - N), (0, 0)))
    bias = [b.reshape(1, D) for b in (b_0, b_1, b_2)]

    degp = _make_seg()(ones_tab, srcs, dsts, zd)
    msg = _make_seg()(h, srcs, dsts, zd)
    h, invb = _tc_layer0(h, msg[0], msg[1], degp[0], degp[1],
                         W_self_0, W_neigh_0, bias[0])

    msg = _make_seg()(h, srcs, dsts, zd)
    h = _tc_layer12(h, msg[0], msg[1], invb, W_self_1, W_neigh_1, bias[1],
                    relu=True)

    msg = _make_seg()(h, srcs, dsts, zd)
    h = _tc_layer12(h, msg[0], msg[1], invb, W_self_2, W_neigh_2, bias[2],
                    relu=False)
    return h[:N]
